# Initial kernel scaffold; baseline (speedup 1.0000x reference)
import jax
import jax.numpy as jnp
from jax.experimental import pallas as pl


def kernel(features, distribution, boxes, obj_embed_w, bn0_g, bn0_b, w_pos, b_pos, w1, b1, bn1_g, bn1_b, w2, b2):
    raise NotImplementedError("write your pallas kernel here")


# speedup vs baseline: 1.4659x; 1.4659x over previous
import functools

import jax
import jax.numpy as jnp
from jax.experimental import pallas as pl

_EPS = 1e-5


def _prep_kernel(cs_in_ref, g0_ref, b0_ref, emb_ref, w1b_ref, stats_ref, wef_ref):
    x = cs_in_ref[...]
    b12 = x[:, 0:2]
    b34 = x[:, 2:4]
    wh = b34 - b12 + 1.0
    c = b12 + 0.5 * wh
    cs = jnp.concatenate([c, wh], axis=1)
    mu = jnp.mean(cs, axis=0, keepdims=True)
    var = jnp.mean((cs - mu) ** 2, axis=0, keepdims=True)
    scale = g0_ref[...] * jax.lax.rsqrt(var + _EPS)
    shift = b0_ref[...] - mu * scale
    stats_ref[...] = jnp.concatenate([scale, shift], axis=0)
    wef = jnp.dot(emb_ref[...].astype(jnp.bfloat16),
                  w1b_ref[...].astype(jnp.bfloat16),
                  preferred_element_type=jnp.float32)
    wef_ref[...] = wef.astype(jnp.bfloat16)


def _main_kernel(feat_ref, dist_ref, cs_in_ref, stats_ref, wpos_ref, bpos_ref,
                 wef_ref, w1a_ref, w1c_ref, b1_ref, h_ref, sums_ref):
    x = cs_in_ref[...]
    b12 = x[:, 0:2]
    b34 = x[:, 2:4]
    wh = b34 - b12 + 1.0
    c = b12 + 0.5 * wh
    cs = jnp.concatenate([c, wh], axis=1)
    scale0 = stats_ref[0:1, :]
    shift0 = stats_ref[1:2, :]
    csn = cs * scale0 + shift0
    pe = jnp.dot(csn.astype(jnp.bfloat16), wpos_ref[...],
                 preferred_element_type=jnp.float32) + bpos_ref[...]
    pe = jnp.maximum(pe, 0.0)

    h = jnp.dot(feat_ref[...].astype(jnp.bfloat16), w1a_ref[...],
                preferred_element_type=jnp.float32)
    h = h + jnp.dot(dist_ref[...].astype(jnp.bfloat16), wef_ref[...],
                    preferred_element_type=jnp.float32)
    h = h + jnp.dot(pe.astype(jnp.bfloat16), w1c_ref[...],
                    preferred_element_type=jnp.float32)
    h = h + b1_ref[...]
    h_ref[...] = h

    part = jnp.concatenate([jnp.sum(h, axis=0, keepdims=True),
                            jnp.sum(h * h, axis=0, keepdims=True)], axis=0)

    @pl.when(pl.program_id(0) == 0)
    def _():
        sums_ref[...] = part

    @pl.when(pl.program_id(0) != 0)
    def _():
        sums_ref[...] += part


def _out_kernel(h_ref, sums_ref, g1_ref, b1n_ref, w2_ref, b2_ref, out_ref,
                *, inv_n):
    mean = sums_ref[0:1, :] * inv_n
    ex2 = sums_ref[1:2, :] * inv_n
    var = ex2 - mean * mean
    scale = g1_ref[...] * jax.lax.rsqrt(var + _EPS)
    shift = b1n_ref[...] - mean * scale
    a = jnp.maximum(h_ref[...] * scale + shift, 0.0)
    out_ref[...] = jnp.dot(a.astype(jnp.bfloat16), w2_ref[...],
                           preferred_element_type=jnp.float32) + b2_ref[...]


def kernel(features, distribution, boxes, obj_embed_w, bn0_g, bn0_b,
           w_pos, b_pos, w1, b1, bn1_g, bn1_b, w2, b2):
    n, obj_dim = features.shape
    nc1 = distribution.shape[1]
    emb_dim = obj_embed_w.shape[1]
    pos_dim = w_pos.shape[1]
    hid = w1.shape[1]
    n_out = w2.shape[1]
    f32 = jnp.float32
    bf16 = jnp.bfloat16

    cs_in = boxes[:, 1:5]
    g0 = bn0_g.reshape(1, 4).astype(f32)
    b0 = bn0_b.reshape(1, 4).astype(f32)
    w1a = w1[:obj_dim].astype(bf16)
    w1b = w1[obj_dim:obj_dim + emb_dim]
    w1c = w1[obj_dim + emb_dim:].astype(bf16)
    wpos = w_pos.astype(bf16)
    bpos = b_pos.reshape(1, pos_dim)
    b1r = b1.reshape(1, hid)
    g1 = bn1_g.reshape(1, hid)
    b1n = bn1_b.reshape(1, hid)
    w2b = w2.astype(bf16)
    b2r = b2.reshape(1, n_out)

    stats, wef = pl.pallas_call(
        _prep_kernel,
        grid=(1,),
        in_specs=[
            pl.BlockSpec((n, 4), lambda i: (0, 0)),
            pl.BlockSpec((1, 4), lambda i: (0, 0)),
            pl.BlockSpec((1, 4), lambda i: (0, 0)),
            pl.BlockSpec((nc1, emb_dim), lambda i: (0, 0)),
            pl.BlockSpec((emb_dim, hid), lambda i: (0, 0)),
        ],
        out_specs=[
            pl.BlockSpec((2, 4), lambda i: (0, 0)),
            pl.BlockSpec((nc1, hid), lambda i: (0, 0)),
        ],
        out_shape=[
            jax.ShapeDtypeStruct((2, 4), f32),
            jax.ShapeDtypeStruct((nc1, hid), bf16),
        ],
    )(cs_in, g0, b0, obj_embed_w, w1b)

    tile = 400
    grid = (n // tile,)
    h, sums = pl.pallas_call(
        _main_kernel,
        grid=grid,
        in_specs=[
            pl.BlockSpec((tile, obj_dim), lambda i: (i, 0)),
            pl.BlockSpec((tile, nc1), lambda i: (i, 0)),
            pl.BlockSpec((tile, 4), lambda i: (i, 0)),
            pl.BlockSpec((2, 4), lambda i: (0, 0)),
            pl.BlockSpec((4, pos_dim), lambda i: (0, 0)),
            pl.BlockSpec((1, pos_dim), lambda i: (0, 0)),
            pl.BlockSpec((nc1, hid), lambda i: (0, 0)),
            pl.BlockSpec((obj_dim, hid), lambda i: (0, 0)),
            pl.BlockSpec((pos_dim, hid), lambda i: (0, 0)),
            pl.BlockSpec((1, hid), lambda i: (0, 0)),
        ],
        out_specs=[
            pl.BlockSpec((tile, hid), lambda i: (i, 0)),
            pl.BlockSpec((2, hid), lambda i: (0, 0)),
        ],
        out_shape=[
            jax.ShapeDtypeStruct((n, hid), f32),
            jax.ShapeDtypeStruct((2, hid), f32),
        ],
    )(features, distribution, cs_in, stats, wpos, bpos, wef, w1a, w1c, b1r)

    logits = pl.pallas_call(
        functools.partial(_out_kernel, inv_n=1.0 / n),
        grid=grid,
        in_specs=[
            pl.BlockSpec((tile, hid), lambda i: (i, 0)),
            pl.BlockSpec((2, hid), lambda i: (0, 0)),
            pl.BlockSpec((1, hid), lambda i: (0, 0)),
            pl.BlockSpec((1, hid), lambda i: (0, 0)),
            pl.BlockSpec((hid, n_out), lambda i: (0, 0)),
            pl.BlockSpec((1, n_out), lambda i: (0, 0)),
        ],
        out_specs=pl.BlockSpec((tile, n_out), lambda i: (i, 0)),
        out_shape=jax.ShapeDtypeStruct((n, n_out), f32),
    )(h, sums, g1, b1n, w2b, b2r)

    return logits


# speedup vs baseline: 1.5383x; 1.0494x over previous
import functools

import jax
import jax.numpy as jnp
from jax.experimental import pallas as pl

_EPS = 1e-5


def _prep_kernel(cs_in_ref, g0_ref, b0_ref, emb_ref, w1b_ref, stats_ref, wef_ref):
    x = cs_in_ref[...]
    b12 = x[:, 0:2]
    b34 = x[:, 2:4]
    wh = b34 - b12 + 1.0
    c = b12 + 0.5 * wh
    cs = jnp.concatenate([c, wh], axis=1)
    mu = jnp.mean(cs, axis=0, keepdims=True)
    var = jnp.mean((cs - mu) ** 2, axis=0, keepdims=True)
    scale = g0_ref[...] * jax.lax.rsqrt(var + _EPS)
    shift = b0_ref[...] - mu * scale
    stats_ref[...] = jnp.concatenate([scale, shift], axis=0)
    wef = jnp.dot(emb_ref[...].astype(jnp.bfloat16),
                  w1b_ref[...].astype(jnp.bfloat16),
                  preferred_element_type=jnp.float32)
    wef_ref[...] = wef.astype(jnp.bfloat16)


def _main_kernel(feat_ref, dist_ref, cs_in_ref, stats_ref, wpos_ref, bpos_ref,
                 wef_ref, w1a_ref, w1c_ref, b1_ref, h_ref, sums_ref):
    x = cs_in_ref[...]
    b12 = x[:, 0:2]
    b34 = x[:, 2:4]
    wh = b34 - b12 + 1.0
    c = b12 + 0.5 * wh
    cs = jnp.concatenate([c, wh], axis=1)
    scale0 = stats_ref[0:1, :]
    shift0 = stats_ref[1:2, :]
    csn = cs * scale0 + shift0
    pe = jnp.dot(csn.astype(jnp.bfloat16), wpos_ref[...],
                 preferred_element_type=jnp.float32) + bpos_ref[...]
    pe = jnp.maximum(pe, 0.0)

    h = jnp.dot(feat_ref[...].astype(jnp.bfloat16), w1a_ref[...],
                preferred_element_type=jnp.float32)
    h = h + jnp.dot(dist_ref[...].astype(jnp.bfloat16), wef_ref[...],
                    preferred_element_type=jnp.float32)
    h = h + jnp.dot(pe.astype(jnp.bfloat16), w1c_ref[...],
                    preferred_element_type=jnp.float32)
    h = h + b1_ref[...]
    h_ref[...] = h.astype(jnp.bfloat16)

    part = jnp.concatenate([jnp.sum(h, axis=0, keepdims=True),
                            jnp.sum(h * h, axis=0, keepdims=True)], axis=0)

    @pl.when(pl.program_id(0) == 0)
    def _():
        sums_ref[...] = part

    @pl.when(pl.program_id(0) != 0)
    def _():
        sums_ref[...] += part


def _out_kernel(h_ref, sums_ref, g1_ref, b1n_ref, w2_ref, b2_ref, out_ref,
                *, inv_n):
    mean = sums_ref[0:1, :] * inv_n
    ex2 = sums_ref[1:2, :] * inv_n
    var = ex2 - mean * mean
    scale = g1_ref[...] * jax.lax.rsqrt(var + _EPS)
    shift = b1n_ref[...] - mean * scale
    a = jnp.maximum(h_ref[...].astype(jnp.float32) * scale + shift, 0.0)
    out_ref[...] = jnp.dot(a.astype(jnp.bfloat16), w2_ref[...],
                           preferred_element_type=jnp.float32) + b2_ref[...]


def kernel(features, distribution, boxes, obj_embed_w, bn0_g, bn0_b,
           w_pos, b_pos, w1, b1, bn1_g, bn1_b, w2, b2):
    n, obj_dim = features.shape
    nc1 = distribution.shape[1]
    emb_dim = obj_embed_w.shape[1]
    pos_dim = w_pos.shape[1]
    hid = w1.shape[1]
    n_out = w2.shape[1]
    f32 = jnp.float32
    bf16 = jnp.bfloat16

    cs_in = boxes[:, 1:5]
    g0 = bn0_g.reshape(1, 4).astype(f32)
    b0 = bn0_b.reshape(1, 4).astype(f32)
    w1a = w1[:obj_dim].astype(bf16)
    w1b = w1[obj_dim:obj_dim + emb_dim]
    w1c = w1[obj_dim + emb_dim:].astype(bf16)
    wpos = w_pos.astype(bf16)
    bpos = b_pos.reshape(1, pos_dim)
    b1r = b1.reshape(1, hid)
    g1 = bn1_g.reshape(1, hid)
    b1n = bn1_b.reshape(1, hid)
    w2b = w2.astype(bf16)
    b2r = b2.reshape(1, n_out)

    stats, wef = pl.pallas_call(
        _prep_kernel,
        grid=(1,),
        in_specs=[
            pl.BlockSpec((n, 4), lambda i: (0, 0)),
            pl.BlockSpec((1, 4), lambda i: (0, 0)),
            pl.BlockSpec((1, 4), lambda i: (0, 0)),
            pl.BlockSpec((nc1, emb_dim), lambda i: (0, 0)),
            pl.BlockSpec((emb_dim, hid), lambda i: (0, 0)),
        ],
        out_specs=[
            pl.BlockSpec((2, 4), lambda i: (0, 0)),
            pl.BlockSpec((nc1, hid), lambda i: (0, 0)),
        ],
        out_shape=[
            jax.ShapeDtypeStruct((2, 4), f32),
            jax.ShapeDtypeStruct((nc1, hid), bf16),
        ],
    )(cs_in, g0, b0, obj_embed_w, w1b)

    tile = 400
    grid = (n // tile,)
    h, sums = pl.pallas_call(
        _main_kernel,
        grid=grid,
        in_specs=[
            pl.BlockSpec((tile, obj_dim), lambda i: (i, 0)),
            pl.BlockSpec((tile, nc1), lambda i: (i, 0)),
            pl.BlockSpec((tile, 4), lambda i: (i, 0)),
            pl.BlockSpec((2, 4), lambda i: (0, 0)),
            pl.BlockSpec((4, pos_dim), lambda i: (0, 0)),
            pl.BlockSpec((1, pos_dim), lambda i: (0, 0)),
            pl.BlockSpec((nc1, hid), lambda i: (0, 0)),
            pl.BlockSpec((obj_dim, hid), lambda i: (0, 0)),
            pl.BlockSpec((pos_dim, hid), lambda i: (0, 0)),
            pl.BlockSpec((1, hid), lambda i: (0, 0)),
        ],
        out_specs=[
            pl.BlockSpec((tile, hid), lambda i: (i, 0)),
            pl.BlockSpec((2, hid), lambda i: (0, 0)),
        ],
        out_shape=[
            jax.ShapeDtypeStruct((n, hid), bf16),
            jax.ShapeDtypeStruct((2, hid), f32),
        ],
    )(features, distribution, cs_in, stats, wpos, bpos, wef, w1a, w1c, b1r)

    logits = pl.pallas_call(
        functools.partial(_out_kernel, inv_n=1.0 / n),
        grid=grid,
        in_specs=[
            pl.BlockSpec((tile, hid), lambda i: (i, 0)),
            pl.BlockSpec((2, hid), lambda i: (0, 0)),
            pl.BlockSpec((1, hid), lambda i: (0, 0)),
            pl.BlockSpec((1, hid), lambda i: (0, 0)),
            pl.BlockSpec((hid, n_out), lambda i: (0, 0)),
            pl.BlockSpec((1, n_out), lambda i: (0, 0)),
        ],
        out_specs=pl.BlockSpec((tile, n_out), lambda i: (i, 0)),
        out_shape=jax.ShapeDtypeStruct((n, n_out), f32),
    )(h, sums, g1, b1n, w2b, b2r)

    return logits


# speedup vs baseline: 1.6718x; 1.0868x over previous
import functools

import jax
import jax.numpy as jnp
from jax.experimental import pallas as pl

_EPS = 1e-5


def _prep_kernel(cs_in_ref, g0_ref, b0_ref, emb_ref, w1b_ref, stats_ref, wef_ref):
    x = cs_in_ref[...]
    b12 = x[:, 0:2]
    b34 = x[:, 2:4]
    wh = b34 - b12 + 1.0
    c = b12 + 0.5 * wh
    cs = jnp.concatenate([c, wh], axis=1)
    mu = jnp.mean(cs, axis=0, keepdims=True)
    var = jnp.mean((cs - mu) ** 2, axis=0, keepdims=True)
    scale = g0_ref[...] * jax.lax.rsqrt(var + _EPS)
    shift = b0_ref[...] - mu * scale
    stats_ref[...] = jnp.concatenate([scale, shift], axis=0)
    wef = jnp.dot(emb_ref[...].astype(jnp.bfloat16),
                  w1b_ref[...].astype(jnp.bfloat16),
                  preferred_element_type=jnp.float32)
    wef_ref[...] = wef.astype(jnp.bfloat16)


def _main_kernel(feat_ref, dist_ref, cs_in_ref, stats_ref, wpos_ref, bpos_ref,
                 wef_ref, w1a_ref, w1c_ref, b1_ref, h_ref, sums_ref):
    x = cs_in_ref[...]
    b12 = x[:, 0:2]
    b34 = x[:, 2:4]
    wh = b34 - b12 + 1.0
    c = b12 + 0.5 * wh
    cs = jnp.concatenate([c, wh], axis=1)
    scale0 = stats_ref[0:1, :]
    shift0 = stats_ref[1:2, :]
    csn = cs * scale0 + shift0
    pe = jnp.dot(csn.astype(jnp.bfloat16), wpos_ref[...],
                 preferred_element_type=jnp.float32) + bpos_ref[...]
    pe = jnp.maximum(pe, 0.0)

    h = jnp.dot(feat_ref[...].astype(jnp.bfloat16), w1a_ref[...],
                preferred_element_type=jnp.float32)
    h = h + jnp.dot(dist_ref[...].astype(jnp.bfloat16), wef_ref[...],
                    preferred_element_type=jnp.float32)
    h = h + jnp.dot(pe.astype(jnp.bfloat16), w1c_ref[...],
                    preferred_element_type=jnp.float32)
    h = h + b1_ref[...]
    h_ref[...] = h.astype(jnp.bfloat16)

    part = jnp.concatenate([jnp.sum(h, axis=0, keepdims=True),
                            jnp.sum(h * h, axis=0, keepdims=True)], axis=0)

    @pl.when(pl.program_id(0) == 0)
    def _():
        sums_ref[...] = part

    @pl.when(pl.program_id(0) != 0)
    def _():
        sums_ref[...] += part


def _out_kernel(h_ref, sums_ref, g1_ref, b1n_ref, w2_ref, b2_ref, out_ref,
                *, inv_n):
    mean = sums_ref[0:1, :] * inv_n
    ex2 = sums_ref[1:2, :] * inv_n
    var = ex2 - mean * mean
    scale = g1_ref[...] * jax.lax.rsqrt(var + _EPS)
    shift = b1n_ref[...] - mean * scale
    a = jnp.maximum(h_ref[...].astype(jnp.float32) * scale + shift, 0.0)
    out_ref[...] = jnp.dot(a.astype(jnp.bfloat16), w2_ref[...],
                           preferred_element_type=jnp.float32) + b2_ref[...]


def kernel(features, distribution, boxes, obj_embed_w, bn0_g, bn0_b,
           w_pos, b_pos, w1, b1, bn1_g, bn1_b, w2, b2):
    n, obj_dim = features.shape
    nc1 = distribution.shape[1]
    emb_dim = obj_embed_w.shape[1]
    pos_dim = w_pos.shape[1]
    hid = w1.shape[1]
    n_out = w2.shape[1]
    f32 = jnp.float32
    bf16 = jnp.bfloat16

    cs_in = boxes[:, 1:5]
    g0 = bn0_g.reshape(1, 4).astype(f32)
    b0 = bn0_b.reshape(1, 4).astype(f32)
    w1a = w1[:obj_dim].astype(bf16)
    w1b = w1[obj_dim:obj_dim + emb_dim]
    w1c = w1[obj_dim + emb_dim:].astype(bf16)
    wpos = w_pos.astype(bf16)
    bpos = b_pos.reshape(1, pos_dim)
    b1r = b1.reshape(1, hid)
    g1 = bn1_g.reshape(1, hid)
    b1n = bn1_b.reshape(1, hid)
    w2b = w2.astype(bf16)
    b2r = b2.reshape(1, n_out)

    stats, wef = pl.pallas_call(
        _prep_kernel,
        grid=(1,),
        in_specs=[
            pl.BlockSpec((n, 4), lambda i: (0, 0)),
            pl.BlockSpec((1, 4), lambda i: (0, 0)),
            pl.BlockSpec((1, 4), lambda i: (0, 0)),
            pl.BlockSpec((nc1, emb_dim), lambda i: (0, 0)),
            pl.BlockSpec((emb_dim, hid), lambda i: (0, 0)),
        ],
        out_specs=[
            pl.BlockSpec((2, 4), lambda i: (0, 0)),
            pl.BlockSpec((nc1, hid), lambda i: (0, 0)),
        ],
        out_shape=[
            jax.ShapeDtypeStruct((2, 4), f32),
            jax.ShapeDtypeStruct((nc1, hid), bf16),
        ],
    )(cs_in, g0, b0, obj_embed_w, w1b)

    tile = 800
    grid = (n // tile,)
    h, sums = pl.pallas_call(
        _main_kernel,
        grid=grid,
        in_specs=[
            pl.BlockSpec((tile, obj_dim), lambda i: (i, 0)),
            pl.BlockSpec((tile, nc1), lambda i: (i, 0)),
            pl.BlockSpec((tile, 4), lambda i: (i, 0)),
            pl.BlockSpec((2, 4), lambda i: (0, 0)),
            pl.BlockSpec((4, pos_dim), lambda i: (0, 0)),
            pl.BlockSpec((1, pos_dim), lambda i: (0, 0)),
            pl.BlockSpec((nc1, hid), lambda i: (0, 0)),
            pl.BlockSpec((obj_dim, hid), lambda i: (0, 0)),
            pl.BlockSpec((pos_dim, hid), lambda i: (0, 0)),
            pl.BlockSpec((1, hid), lambda i: (0, 0)),
        ],
        out_specs=[
            pl.BlockSpec((tile, hid), lambda i: (i, 0)),
            pl.BlockSpec((2, hid), lambda i: (0, 0)),
        ],
        out_shape=[
            jax.ShapeDtypeStruct((n, hid), bf16),
            jax.ShapeDtypeStruct((2, hid), f32),
        ],
    )(features, distribution, cs_in, stats, wpos, bpos, wef, w1a, w1c, b1r)

    logits = pl.pallas_call(
        functools.partial(_out_kernel, inv_n=1.0 / n),
        grid=grid,
        in_specs=[
            pl.BlockSpec((tile, hid), lambda i: (i, 0)),
            pl.BlockSpec((2, hid), lambda i: (0, 0)),
            pl.BlockSpec((1, hid), lambda i: (0, 0)),
            pl.BlockSpec((1, hid), lambda i: (0, 0)),
            pl.BlockSpec((hid, n_out), lambda i: (0, 0)),
            pl.BlockSpec((1, n_out), lambda i: (0, 0)),
        ],
        out_specs=pl.BlockSpec((tile, n_out), lambda i: (i, 0)),
        out_shape=jax.ShapeDtypeStruct((n, n_out), f32),
    )(h, sums, g1, b1n, w2b, b2r)

    return logits


# speedup vs baseline: 1.7221x; 1.0301x over previous
import functools

import jax
import jax.numpy as jnp
from jax.experimental import pallas as pl

_EPS = 1e-5


def _prep_kernel(bt_ref, g0_ref, b0_ref, emb_ref, w1b_ref, stats_ref, wef_ref):
    b12_t = bt_ref[1:3, :]
    b34_t = bt_ref[3:5, :]
    wh_t = b34_t - b12_t + 1.0
    c_t = b12_t + 0.5 * wh_t
    cs_t = jnp.concatenate([c_t, wh_t], axis=0)
    mu = jnp.mean(cs_t, axis=1, keepdims=True)
    var = jnp.mean((cs_t - mu) ** 2, axis=1, keepdims=True)
    scale = g0_ref[...] * jax.lax.rsqrt(var + _EPS)
    shift = b0_ref[...] - mu * scale
    stats_ref[...] = jnp.transpose(jnp.concatenate([scale, shift], axis=1))
    wef = jnp.dot(emb_ref[...].astype(jnp.bfloat16),
                  w1b_ref[...].astype(jnp.bfloat16),
                  preferred_element_type=jnp.float32)
    wef_ref[...] = wef.astype(jnp.bfloat16)


def _main_kernel(feat_ref, dist_ref, box_ref, stats_ref, wpos_ref, bpos_ref,
                 wef_ref, w1a_ref, w1c_ref, b1_ref, h_ref, sums_ref):
    x = box_ref[...]
    b12 = x[:, 1:3]
    b34 = x[:, 3:5]
    wh = b34 - b12 + 1.0
    c = b12 + 0.5 * wh
    cs = jnp.concatenate([c, wh], axis=1)
    scale0 = stats_ref[0:1, :]
    shift0 = stats_ref[1:2, :]
    csn = cs * scale0 + shift0
    pe = jnp.dot(csn.astype(jnp.bfloat16), wpos_ref[...],
                 preferred_element_type=jnp.float32) + bpos_ref[...]
    pe = jnp.maximum(pe, 0.0)

    h = jnp.dot(feat_ref[...].astype(jnp.bfloat16), w1a_ref[...],
                preferred_element_type=jnp.float32)
    h = h + jnp.dot(dist_ref[...].astype(jnp.bfloat16), wef_ref[...],
                    preferred_element_type=jnp.float32)
    h = h + jnp.dot(pe.astype(jnp.bfloat16), w1c_ref[...],
                    preferred_element_type=jnp.float32)
    h = h + b1_ref[...]
    h_ref[...] = h.astype(jnp.bfloat16)

    part = jnp.concatenate([jnp.sum(h, axis=0, keepdims=True),
                            jnp.sum(h * h, axis=0, keepdims=True)], axis=0)

    @pl.when(pl.program_id(0) == 0)
    def _():
        sums_ref[...] = part

    @pl.when(pl.program_id(0) != 0)
    def _():
        sums_ref[...] += part


def _out_kernel(h_ref, sums_ref, g1_ref, b1n_ref, w2_ref, b2_ref, out_ref,
                *, inv_n):
    mean = sums_ref[0:1, :] * inv_n
    ex2 = sums_ref[1:2, :] * inv_n
    var = ex2 - mean * mean
    scale = (g1_ref[...] * jax.lax.rsqrt(var + _EPS)).astype(jnp.bfloat16)
    shift = (b1n_ref[...] - mean * (g1_ref[...] * jax.lax.rsqrt(var + _EPS))).astype(jnp.bfloat16)
    a = jnp.maximum(h_ref[...] * scale + shift, jnp.bfloat16(0.0))
    out_ref[...] = jnp.dot(a, w2_ref[...],
                           preferred_element_type=jnp.float32) + b2_ref[...]


def kernel(features, distribution, boxes, obj_embed_w, bn0_g, bn0_b,
           w_pos, b_pos, w1, b1, bn1_g, bn1_b, w2, b2):
    n, obj_dim = features.shape
    nc1 = distribution.shape[1]
    emb_dim = obj_embed_w.shape[1]
    pos_dim = w_pos.shape[1]
    hid = w1.shape[1]
    n_out = w2.shape[1]
    f32 = jnp.float32
    bf16 = jnp.bfloat16

    bt = jnp.transpose(boxes)
    g0 = bn0_g.reshape(4, 1).astype(f32)
    b0 = bn0_b.reshape(4, 1).astype(f32)
    w1a = w1[:obj_dim].astype(bf16)
    w1b = w1[obj_dim:obj_dim + emb_dim]
    w1c = w1[obj_dim + emb_dim:].astype(bf16)
    wpos = w_pos.astype(bf16)
    bpos = b_pos.reshape(1, pos_dim)
    b1r = b1.reshape(1, hid)
    g1 = bn1_g.reshape(1, hid)
    b1n = bn1_b.reshape(1, hid)
    w2b = w2.astype(bf16)
    b2r = b2.reshape(1, n_out)

    stats, wef = pl.pallas_call(
        _prep_kernel,
        grid=(1,),
        in_specs=[
            pl.BlockSpec((5, n), lambda i: (0, 0)),
            pl.BlockSpec((4, 1), lambda i: (0, 0)),
            pl.BlockSpec((4, 1), lambda i: (0, 0)),
            pl.BlockSpec((nc1, emb_dim), lambda i: (0, 0)),
            pl.BlockSpec((emb_dim, hid), lambda i: (0, 0)),
        ],
        out_specs=[
            pl.BlockSpec((2, 4), lambda i: (0, 0)),
            pl.BlockSpec((nc1, hid), lambda i: (0, 0)),
        ],
        out_shape=[
            jax.ShapeDtypeStruct((2, 4), f32),
            jax.ShapeDtypeStruct((nc1, hid), bf16),
        ],
    )(bt, g0, b0, obj_embed_w, w1b)

    tile = 800
    grid = (n // tile,)
    h, sums = pl.pallas_call(
        _main_kernel,
        grid=grid,
        in_specs=[
            pl.BlockSpec((tile, obj_dim), lambda i: (i, 0)),
            pl.BlockSpec((tile, nc1), lambda i: (i, 0)),
            pl.BlockSpec((tile, 5), lambda i: (i, 0)),
            pl.BlockSpec((2, 4), lambda i: (0, 0)),
            pl.BlockSpec((4, pos_dim), lambda i: (0, 0)),
            pl.BlockSpec((1, pos_dim), lambda i: (0, 0)),
            pl.BlockSpec((nc1, hid), lambda i: (0, 0)),
            pl.BlockSpec((obj_dim, hid), lambda i: (0, 0)),
            pl.BlockSpec((pos_dim, hid), lambda i: (0, 0)),
            pl.BlockSpec((1, hid), lambda i: (0, 0)),
        ],
        out_specs=[
            pl.BlockSpec((tile, hid), lambda i: (i, 0)),
            pl.BlockSpec((2, hid), lambda i: (0, 0)),
        ],
        out_shape=[
            jax.ShapeDtypeStruct((n, hid), bf16),
            jax.ShapeDtypeStruct((2, hid), f32),
        ],
    )(features, distribution, boxes, stats, wpos, bpos, wef, w1a, w1c, b1r)

    logits = pl.pallas_call(
        functools.partial(_out_kernel, inv_n=1.0 / n),
        grid=grid,
        in_specs=[
            pl.BlockSpec((tile, hid), lambda i: (i, 0)),
            pl.BlockSpec((2, hid), lambda i: (0, 0)),
            pl.BlockSpec((1, hid), lambda i: (0, 0)),
            pl.BlockSpec((1, hid), lambda i: (0, 0)),
            pl.BlockSpec((hid, n_out), lambda i: (0, 0)),
            pl.BlockSpec((1, n_out), lambda i: (0, 0)),
        ],
        out_specs=pl.BlockSpec((tile, n_out), lambda i: (i, 0)),
        out_shape=jax.ShapeDtypeStruct((n, n_out), f32),
    )(h, sums, g1, b1n, w2b, b2r)

    return logits
